# baseline (speedup 1.0000x reference)
import jax
import jax.numpy as jnp
from jax.experimental import pallas as pl

_B = 64
_RATIO = 0.8
_TEMP = 0.07
_NEG_W = 1.0


def _gconv(x, src, dst, evalid, Wrel, brel, Wroot):
    n = x.shape[0]
    w = evalid.astype(x.dtype)
    agg = jax.ops.segment_sum(x[src] * w[:, None], dst, num_segments=n)
    deg = jax.ops.segment_sum(w, dst, num_segments=n)
    mean = agg / jnp.maximum(deg, 1.0)[:, None]
    return mean @ Wrel + brel + x @ Wroot


def _topk_mask(score, batch, valid, ratio, num_graphs):
    s = jnp.where(valid, score, -1e9)
    order = jnp.lexsort((-s, batch))
    counts = jnp.bincount(batch, length=num_graphs)
    starts = jnp.cumsum(counts) - counts
    bs = batch[order]
    rank = jnp.arange(s.shape[0]) - starts[bs]
    vcnt = jax.ops.segment_sum(valid.astype(jnp.int32), batch, num_segments=num_graphs)
    k = jnp.ceil(ratio * vcnt.astype(jnp.float32)).astype(jnp.int32)
    keep_sorted = rank < k[bs]
    mask = jnp.zeros(s.shape[0], dtype=bool).at[order].set(keep_sorted)
    return mask & valid


def _side(x, ei, batch, p, num_graphs, ratio):
    src, dst = ei[0], ei[1]
    evalid = jnp.ones(src.shape[0], dtype=bool)
    nmask = jnp.ones(x.shape[0], dtype=bool)
    Wrel, brel, Wroot = p["conv0"]
    x = jax.nn.relu(_gconv(x, src, dst, evalid, Wrel, brel, Wroot))
    xs = [jax.ops.segment_sum(x, batch, num_segments=num_graphs)]
    convs = p["convs"]
    for i in range(len(convs)):
        Wrel, brel, Wroot = convs[i]
        x = jax.nn.relu(_gconv(x, src, dst, evalid, Wrel, brel, Wroot))
        x = x * nmask[:, None].astype(x.dtype)
        xs.append(jax.ops.segment_sum(x, batch, num_segments=num_graphs))
        if i % 2 == 0 and i < len(convs) - 1:
            pWrel, pbrel, pWroot = p["pool"]
            score = jnp.tanh(_gconv(x, src, dst, evalid, pWrel, pbrel, pWroot)[:, 0])
            nmask = _topk_mask(score, batch, nmask, ratio, num_graphs)
            x = x * score[:, None] * nmask[:, None].astype(x.dtype)
            evalid = nmask[src] & nmask[dst]
    return jnp.concatenate(xs, axis=1)


def _noop_pallas(h):
    def body(x_ref, o_ref):
        o_ref[...] = x_ref[...]
    return pl.pallas_call(
        body, out_shape=jax.ShapeDtypeStruct(h.shape, h.dtype))(h)


def kernel(x, fc_x, params, edge_index, fc_edge_index, batch):
    sc = _side(x, edge_index, batch, params["sc"], _B, _RATIO)
    fc = _side(fc_x, fc_edge_index, batch, params["fc"], _B, _RATIO)
    h = jnp.concatenate([sc, fc], axis=1)
    h = _noop_pallas(h)
    W1, b1 = params["lin1"]
    W2, b2 = params["lin2"]
    W3, b3 = params["lin3"]
    h = jax.nn.relu(h @ W1 + b1)
    h = jax.nn.relu(h @ W2 + b2)
    logits = h @ W3 + b3
    scn = sc / jnp.maximum(jnp.linalg.norm(sc, axis=1, keepdims=True), 1e-12)
    fcn = fc / jnp.maximum(jnp.linalg.norm(fc, axis=1, keepdims=True), 1e-12)
    l_sc = scn @ fcn.T / _TEMP
    l_fc = fcn @ scn.T / _TEMP
    c_sc = scn @ scn.T / _TEMP
    c_fc = fcn @ fcn.T / _TEMP
    nb = sc.shape[0]
    pos_mask = 1.0 - jnp.eye(nb, dtype=jnp.float32)
    sc_logits = jnp.concatenate([l_sc, _NEG_W * c_sc * pos_mask], axis=1)
    fc_logits = jnp.concatenate([l_fc, _NEG_W * c_fc * pos_mask], axis=1)
    m = jnp.concatenate([jnp.eye(nb, dtype=jnp.float32), jnp.zeros((nb, nb), jnp.float32)], axis=1)
    loss_i = -jnp.log((jax.nn.softmax(sc_logits, axis=1) * m).sum(1))
    loss_t = -jnp.log((jax.nn.softmax(fc_logits, axis=1) * m).sum(1))
    return jax.nn.log_softmax(logits, axis=-1) + (loss_i.mean() + loss_t.mean()) / 2.0


# speedup vs baseline: 3.9992x; 3.9992x over previous
import functools

import jax
import jax.numpy as jnp
from jax import lax
from jax.experimental import pallas as pl
from jax.experimental.pallas import tpu as pltpu
from jax.experimental.pallas import tpu_sc as plsc

_N = 10000
_E = 320000
_NP = 10240
_D = 128
_B = 64
_RATIO = 0.8
_TEMP = 0.07

_NC = 2
_NS = 16
_NW = _NC * _NS
_EPW = _E // _NW
_CH = 80
_NIT = _EPW // _CH
_RPT = _NP // _NS
_ZR = 160

_BLK = 1280
_G = _NP // _BLK


def _edge_agg_body(table, src, dst, out, idx_s, idx_d, rows, zbuf, acc, sem):
    cid = lax.axis_index("c")
    sid = lax.axis_index("s")
    wid = sid * _NC + cid

    z16 = jnp.zeros((16,), jnp.float32)

    def zb(i, c):
        r = i // (_D // 16)
        col = (i % (_D // 16)) * 16
        zbuf[r, pl.ds(col, 16)] = z16
        return c

    lax.fori_loop(0, _ZR * (_D // 16), zb, 0)

    def zc(j, c):
        pltpu.sync_copy(zbuf, acc.at[pl.ds(sid * _RPT + j * _ZR, _ZR)])
        return c

    lax.fori_loop(0, _RPT // _ZR, zc, 0)
    plsc.subcore_barrier()

    base = wid * _EPW

    def step(t, c):
        off = base + t * _CH
        pltpu.sync_copy(src.at[pl.ds(off, _CH)], idx_s)
        pltpu.sync_copy(dst.at[pl.ds(off, _CH)], idx_d)
        pltpu.async_copy(table.at[idx_s], rows, sem).wait()
        pltpu.sync_copy(rows, acc.at[idx_d], add=True)
        return c

    lax.fori_loop(0, _NIT, step, 0)
    plsc.subcore_barrier()
    pltpu.sync_copy(acc.at[pl.ds(sid * _RPT, _RPT)],
                    out.at[cid, pl.ds(sid * _RPT, _RPT)])


@functools.cache
def _edge_agg_kernel():
    return pl.kernel(
        _edge_agg_body,
        mesh=plsc.VectorSubcoreMesh(core_axis_name="c",
                                    subcore_axis_name="s"),
        out_type=jax.ShapeDtypeStruct((_NC, _NP, _D), jnp.float32),
        scratch_types=[
            pltpu.VMEM((_CH,), jnp.int32),
            pltpu.VMEM((_CH,), jnp.int32),
            pltpu.VMEM((_CH, _D), jnp.float32),
            pltpu.VMEM((_ZR, _D), jnp.float32),
            pltpu.VMEM_SHARED((_NP, _D), jnp.float32),
            pltpu.SemaphoreType.DMA,
        ],
    )


def _edge_agg(tab, src, dst):
    return _edge_agg_kernel()(tab, src, dst)


def _conv_body(acc_ref, dacc_ref, tab_ref, wrel_ref, brel_ref, wroot_ref,
               nm_ref, batch_ref, out_tab_ref, ro_ref):
    agg = acc_ref[0] + acc_ref[1]
    deg = dacc_ref[0, :, 0:1] + dacc_ref[1, :, 0:1]
    x = tab_ref[...]
    nm = nm_ref[...]
    mean = agg / jnp.maximum(deg, 1.0)
    h = (jnp.dot(mean, wrel_ref[...], preferred_element_type=jnp.float32)
         + brel_ref[...]
         + jnp.dot(x, wroot_ref[...], preferred_element_type=jnp.float32))
    h = jnp.maximum(h, 0.0) * nm
    out_tab_ref[...] = h
    b = batch_ref[...]
    g = lax.broadcasted_iota(jnp.int32, (_BLK, _B), 1).astype(jnp.float32)
    oh = (b == g).astype(jnp.float32)
    contrib = lax.dot_general(oh, h, (((0,), (0,)), ((), ())),
                              preferred_element_type=jnp.float32)

    @pl.when(pl.program_id(0) == 0)
    def _():
        ro_ref[...] = contrib

    @pl.when(pl.program_id(0) != 0)
    def _():
        ro_ref[...] += contrib


def _conv_step(acc, dacc, tab, wrel, brel, wroot, nm_col, batch_col):
    return pl.pallas_call(
        _conv_body,
        grid=(_G,),
        in_specs=[
            pl.BlockSpec((_NC, _BLK, _D), lambda i: (0, i, 0)),
            pl.BlockSpec((_NC, _BLK, _D), lambda i: (0, i, 0)),
            pl.BlockSpec((_BLK, _D), lambda i: (i, 0)),
            pl.BlockSpec((128, 128), lambda i: (0, 0)),
            pl.BlockSpec((1, 128), lambda i: (0, 0)),
            pl.BlockSpec((128, 128), lambda i: (0, 0)),
            pl.BlockSpec((_BLK, 1), lambda i: (i, 0)),
            pl.BlockSpec((_BLK, 1), lambda i: (i, 0)),
        ],
        out_specs=[
            pl.BlockSpec((_BLK, _D), lambda i: (i, 0)),
            pl.BlockSpec((_B, 128), lambda i: (0, 0)),
        ],
        out_shape=[
            jax.ShapeDtypeStruct((_NP, _D), jnp.float32),
            jax.ShapeDtypeStruct((_B, 128), jnp.float32),
        ],
    )(acc, dacc, tab, wrel, brel, wroot, nm_col, batch_col)


def _pool_body(acc_ref, dacc_ref, tab_ref, pwrel_ref, pbrel_ref, pwroot_ref,
               out_ref):
    agg = acc_ref[0] + acc_ref[1]
    deg = dacc_ref[0, :, 0:1] + dacc_ref[1, :, 0:1]
    x = tab_ref[...]
    mean = agg / jnp.maximum(deg, 1.0)
    s = (jnp.dot(mean, pwrel_ref[...], preferred_element_type=jnp.float32)
         + pbrel_ref[...]
         + jnp.dot(x, pwroot_ref[...], preferred_element_type=jnp.float32))
    out_ref[...] = jnp.tanh(s)


def _pool_score(acc, dacc, tab, pwrel, pbrel, pwroot):
    return pl.pallas_call(
        _pool_body,
        grid=(_G,),
        in_specs=[
            pl.BlockSpec((_NC, _BLK, _D), lambda i: (0, i, 0)),
            pl.BlockSpec((_NC, _BLK, _D), lambda i: (0, i, 0)),
            pl.BlockSpec((_BLK, _D), lambda i: (i, 0)),
            pl.BlockSpec((128, 128), lambda i: (0, 0)),
            pl.BlockSpec((1, 128), lambda i: (0, 0)),
            pl.BlockSpec((128, 128), lambda i: (0, 0)),
        ],
        out_specs=pl.BlockSpec((_BLK, 128), lambda i: (i, 0)),
        out_shape=jax.ShapeDtypeStruct((_NP, 128), jnp.float32),
    )(acc, dacc, tab, pwrel, pbrel, pwroot)


def _topk_body(scol_ref, srow_ref, bcol_ref, brow_ref, tab_ref,
               out_tab_ref, out_msk_ref):
    i = pl.program_id(0)
    si = scol_ref[pl.ds(i * _BLK, _BLK), :]
    bi = bcol_ref[pl.ds(i * _BLK, _BLK), :]
    ball = bcol_ref[...]
    gall = lax.broadcasted_iota(jnp.int32, (_NP, _B), 1).astype(jnp.float32)
    counts = jnp.sum((ball == gall).astype(jnp.float32), axis=0,
                     keepdims=True)
    krow = jnp.ceil(_RATIO * counts)
    ohi = (bi == lax.broadcasted_iota(jnp.int32, (_BLK, _B), 1)
           .astype(jnp.float32)).astype(jnp.float32)
    kv = lax.dot_general(ohi, krow, (((1,), (1,)), ((), ())),
                         preferred_element_type=jnp.float32)
    irow = (lax.broadcasted_iota(jnp.int32, (_BLK, _BLK), 0)
            .astype(jnp.float32) + (i * _BLK).astype(jnp.float32))

    def col_chunk(c, cnt):
        sj = srow_ref[:, pl.ds(c * _BLK, _BLK)]
        bj = brow_ref[:, pl.ds(c * _BLK, _BLK)]
        jcol = (lax.broadcasted_iota(jnp.int32, (_BLK, _BLK), 1)
                .astype(jnp.float32) + (c * _BLK).astype(jnp.float32))
        same = bi == bj
        ahead = (sj > si) | ((sj == si) & (jcol < irow))
        add = (same & ahead).astype(jnp.float32)
        return cnt + jnp.sum(add, axis=1, keepdims=True)

    cnt = lax.fori_loop(0, _G, col_chunk, jnp.zeros((_BLK, 1), jnp.float32))
    keep = (cnt < kv).astype(jnp.float32)
    out_tab_ref[...] = tab_ref[...] * si * keep
    out_msk_ref[...] = jnp.concatenate(
        [keep, jnp.zeros((_BLK, 127), jnp.float32)], axis=1)


def _topk_apply(scol, srow, bcol, brow, tab):
    return pl.pallas_call(
        _topk_body,
        grid=(_G,),
        in_specs=[
            pl.BlockSpec((_NP, 1), lambda i: (0, 0)),
            pl.BlockSpec((1, _NP), lambda i: (0, 0)),
            pl.BlockSpec((_NP, 1), lambda i: (0, 0)),
            pl.BlockSpec((1, _NP), lambda i: (0, 0)),
            pl.BlockSpec((_BLK, _D), lambda i: (i, 0)),
        ],
        out_specs=[
            pl.BlockSpec((_BLK, _D), lambda i: (i, 0)),
            pl.BlockSpec((_BLK, 128), lambda i: (i, 0)),
        ],
        out_shape=[
            jax.ShapeDtypeStruct((_NP, _D), jnp.float32),
            jax.ShapeDtypeStruct((_NP, 128), jnp.float32),
        ],
    )(scol, srow, bcol, brow, tab)


def _head_body(sc_ref, fc_ref, w1a_ref, w1b_ref, b1_ref, w2_ref, b2_ref,
               w3_ref, b3_ref, out_ref):
    sc = sc_ref[...]
    fc = fc_ref[...]
    h = jnp.maximum(
        jnp.dot(sc, w1a_ref[...], preferred_element_type=jnp.float32)
        + jnp.dot(fc, w1b_ref[...], preferred_element_type=jnp.float32)
        + b1_ref[...], 0.0)
    h = jnp.maximum(
        jnp.dot(h, w2_ref[...], preferred_element_type=jnp.float32)
        + b2_ref[...], 0.0)
    logits = (jnp.dot(h, w3_ref[...], preferred_element_type=jnp.float32)
              + b3_ref[...])

    def nrm(v):
        n2 = jnp.sum(v * v, axis=1, keepdims=True)
        return v / jnp.maximum(jnp.sqrt(n2), 1e-12)

    scn = nrm(sc)
    fcn = nrm(fc)
    dims = (((1,), (1,)), ((), ()))
    l_sc = lax.dot_general(scn, fcn, dims,
                           preferred_element_type=jnp.float32) / _TEMP
    l_fc = lax.dot_general(fcn, scn, dims,
                           preferred_element_type=jnp.float32) / _TEMP
    c_sc = lax.dot_general(scn, scn, dims,
                           preferred_element_type=jnp.float32) / _TEMP
    c_fc = lax.dot_general(fcn, fcn, dims,
                           preferred_element_type=jnp.float32) / _TEMP
    eye = (lax.broadcasted_iota(jnp.int32, (_B, _B), 0)
           == lax.broadcasted_iota(jnp.int32, (_B, _B), 1)
           ).astype(jnp.float32)
    neg_sc = c_sc * (1.0 - eye)
    neg_fc = c_fc * (1.0 - eye)

    def nce(l, neg):
        m = jnp.maximum(jnp.max(l, axis=1, keepdims=True),
                        jnp.max(neg, axis=1, keepdims=True))
        ssum = (jnp.sum(jnp.exp(l - m), axis=1, keepdims=True)
                + jnp.sum(jnp.exp(neg - m), axis=1, keepdims=True))
        lse = m + jnp.log(ssum)
        diag = jnp.sum(l * eye, axis=1, keepdims=True)
        return jnp.sum(lse - diag) / _B

    loss = (nce(l_sc, neg_sc) + nce(l_fc, neg_fc)) * 0.5
    lm = jnp.max(logits, axis=1, keepdims=True)
    lz = logits - lm
    lse2 = jnp.log(jnp.sum(jnp.exp(lz), axis=1, keepdims=True))
    out_ref[...] = lz - lse2 + loss


def _head(scf, fcf, w1a, w1b, b1, w2, b2, w3, b3):
    return pl.pallas_call(
        _head_body,
        out_shape=jax.ShapeDtypeStruct((_B, 10), jnp.float32),
    )(scf, fcf, w1a, w1b, b1, w2, b2, w3, b3)


def _pad_rows(xin):
    return jnp.concatenate(
        [xin, jnp.zeros((_NP - _N, _D), jnp.float32)], axis=0)


def _side(xin, ei, batch_col, p):
    src, dst = ei[0], ei[1]
    tab = _pad_rows(xin)
    ones_tab = jnp.concatenate(
        [jnp.ones((_NP, 1), jnp.float32),
         jnp.zeros((_NP, 127), jnp.float32)], axis=1)
    ones_col = jnp.ones((_NP, 1), jnp.float32)

    dacc_full = _edge_agg(ones_tab, src, dst)

    wr, br, wt = p["conv0"]
    a = _edge_agg(tab, src, dst)
    tab, r0 = _conv_step(a, dacc_full, tab, wr, br.reshape(1, 128), wt,
                         ones_col, batch_col)
    wr, br, wt = p["convs"][0]
    a = _edge_agg(tab, src, dst)
    tab, r1 = _conv_step(a, dacc_full, tab, wr, br.reshape(1, 128), wt,
                         ones_col, batch_col)
    pw, pb, pwt = p["pool"]
    pwf = jnp.pad(pw, ((0, 0), (0, 127)))
    pbf = jnp.pad(pb.reshape(1, 1), ((0, 0), (0, 127)))
    pwtf = jnp.pad(pwt, ((0, 0), (0, 127)))
    a = _edge_agg(tab, src, dst)
    sfull = _pool_score(a, dacc_full, tab, pwf, pbf, pwtf)
    scol = sfull[:, :1]
    tab, msk_tab = _topk_apply(scol, scol.reshape(1, _NP), batch_col,
                               batch_col.reshape(1, _NP), tab)
    nm_col = msk_tab[:, :1]

    dacc_msk = _edge_agg(msk_tab, src, dst)

    wr, br, wt = p["convs"][1]
    a = _edge_agg(tab, src, dst)
    tab, r2 = _conv_step(a, dacc_msk, tab, wr, br.reshape(1, 128), wt,
                         nm_col, batch_col)
    wr, br, wt = p["convs"][2]
    a = _edge_agg(tab, src, dst)
    tab, r3 = _conv_step(a, dacc_msk, tab, wr, br.reshape(1, 128), wt,
                         nm_col, batch_col)
    return jnp.concatenate([r0, r1, r2, r3], axis=1)


def kernel(x, fc_x, params, edge_index, fc_edge_index, batch):
    batch_col = jnp.concatenate(
        [batch.astype(jnp.float32),
         jnp.full((_NP - _N,), -1.0, jnp.float32)]).reshape(_NP, 1)
    scf = _side(x, edge_index, batch_col, params["sc"])
    fcf = _side(fc_x, fc_edge_index, batch_col, params["fc"])
    w1, b1 = params["lin1"]
    w2, b2 = params["lin2"]
    w3, b3 = params["lin3"]
    return _head(scf, fcf, w1[:512], w1[512:], b1.reshape(1, 128),
                 w2, b2.reshape(1, 64), w3, b3.reshape(1, 10))


# speedup vs baseline: 9.5130x; 2.3788x over previous
import functools

import jax
import jax.numpy as jnp
from jax import lax
from jax.experimental import pallas as pl
from jax.experimental.pallas import tpu as pltpu
from jax.experimental.pallas import tpu_sc as plsc

_N = 10000
_E = 320000
_NP = 10240
_D = 128
_B = 64
_RATIO = 0.8
_TEMP = 0.07

_NC = 2
_NS = 16
_NW = _NC * _NS
_EPW = _E // _NW
_CH = 80
_NIT = _EPW // _CH
_DCH = 80
_DNIT = _EPW // _DCH
_RPT = _NP // _NS
_ZR = 16

_BLK = 1280
_G = _NP // _BLK


def _zero_acc_slice(zbuf, acc, sid):
    z16 = jnp.zeros((16,), jnp.float32)

    def zb(i, c):
        r = i // (_D // 16)
        col = (i % (_D // 16)) * 16
        zbuf[r, pl.ds(col, 16)] = z16
        return c

    lax.fori_loop(0, _ZR * (_D // 16), zb, 0)

    def zc(j, c):
        pltpu.sync_copy(zbuf, acc.at[pl.ds(sid * _RPT + j * _ZR, _ZR)])
        return c

    lax.fori_loop(0, _RPT // _ZR, zc, 0)


def _edge_agg_body(table, src, dst, out,
                   is0, is1, is2, is3, id0, id1, id2, id3,
                   rows0, rows1, zbuf, acc,
                   gs0, gs1, ii0, ii1, ii2, ii3):
    cid = lax.axis_index("c")
    sid = lax.axis_index("s")
    wid = sid * _NC + cid
    base = wid * _EPW

    isl = [is0, is1, is2, is3]
    idl = [id0, id1, id2, id3]
    iil = [ii0, ii1, ii2, ii3]
    rl = [rows0, rows1]
    gl = [gs0, gs1]

    def idx_fire(j, t):
        pltpu.async_copy(src.at[pl.ds(base + t * _CH, _CH)], isl[j], iil[j])
        pltpu.async_copy(dst.at[pl.ds(base + t * _CH, _CH)], idl[j], iil[j])

    def idx_wait(j, t):
        pltpu.make_async_copy(
            src.at[pl.ds(base + t * _CH, _CH)], isl[j], iil[j]).wait()
        pltpu.make_async_copy(
            dst.at[pl.ds(base + t * _CH, _CH)], idl[j], iil[j]).wait()

    _zero_acc_slice(zbuf, acc, sid)
    plsc.subcore_barrier()

    for j in range(4):
        idx_fire(j, j)
    idx_wait(0, 0)
    pltpu.async_copy(table.at[is0], rows0, gs0)
    idx_wait(1, 1)
    pltpu.async_copy(table.at[is1], rows1, gs1)

    def body(u, c):
        for j in range(4):
            t = 4 * u + j
            b = j % 2
            pltpu.make_async_copy(table.at[isl[j]], rl[b], gl[b]).wait()
            pltpu.sync_copy(rl[b], acc.at[idl[j]], add=True)

            @pl.when(t + 4 <= _NIT - 1)
            def _():
                idx_fire(j, t + 4)

            @pl.when(t + 2 <= _NIT - 1)
            def _():
                idx_wait((j + 2) % 4, t + 2)
                pltpu.async_copy(table.at[isl[(j + 2) % 4]], rl[b], gl[b])

        return c

    lax.fori_loop(0, (_NIT - 1) // 4, body, 0)
    tl = _NIT - 1
    pltpu.make_async_copy(table.at[is0], rows0, gs0).wait()
    pltpu.sync_copy(rows0, acc.at[idl[tl % 4]], add=True)
    plsc.subcore_barrier()
    pltpu.sync_copy(acc.at[pl.ds(sid * _RPT, _RPT)],
                    out.at[cid, pl.ds(sid * _RPT, _RPT)])


@functools.cache
def _edge_agg_kernel():
    return pl.kernel(
        _edge_agg_body,
        mesh=plsc.VectorSubcoreMesh(core_axis_name="c",
                                    subcore_axis_name="s"),
        out_type=jax.ShapeDtypeStruct((_NC, _NP, _D), jnp.float32),
        scratch_types=(
            [pltpu.VMEM((_CH,), jnp.int32) for _ in range(8)]
            + [pltpu.VMEM((_CH, _D), jnp.float32) for _ in range(2)]
            + [pltpu.VMEM((_ZR, _D), jnp.float32),
               pltpu.VMEM_SHARED((_NP, _D), jnp.float32)]
            + [pltpu.SemaphoreType.DMA for _ in range(6)]),
    )


def _edge_agg(tab, src, dst):
    return _edge_agg_kernel()(tab, src, dst)


def _deg_agg_body(mvec_hbm, src4, dst4, out, srcb, dstb, mvec, degacc, sumb,
                  outb, stage):
    cid = lax.axis_index("c")
    sid = lax.axis_index("s")
    wid = sid * _NC + cid

    pltpu.sync_copy(src4.at[wid], srcb)
    pltpu.sync_copy(dst4.at[wid], dstb)
    pltpu.sync_copy(mvec_hbm, mvec)
    z16 = jnp.zeros((16,), jnp.float32)

    def zd(i, c):
        degacc[0, pl.ds(i * 16, 16)] = z16
        return c

    lax.fori_loop(0, _NP // 16, zd, 0)

    def step(t, c):
        for k in range(_DCH // 16):
            s16 = srcb[t, pl.ds(k * 16, 16)]
            d16 = dstb[t, pl.ds(k * 16, 16)]
            vals = plsc.load_gather(mvec, [s16])
            plsc.addupdate_scatter(degacc, [jnp.zeros((16,), jnp.int32),
                                            d16], vals)
        return c

    lax.fori_loop(0, _DNIT, step, 0)
    pltpu.sync_copy(degacc, stage.at[pl.ds(sid, 1)])
    plsc.subcore_barrier()
    for r in range(_NS):
        pltpu.sync_copy(stage.at[pl.ds(r, 1), pl.ds(sid * _RPT, _RPT)],
                        sumb.at[pl.ds(r, 1)])

    def red(j, c):
        a = sumb[0, pl.ds(j * 16, 16)]
        for r in range(1, _NS):
            a = a + sumb[r, pl.ds(j * 16, 16)]
        outb[pl.ds(j * 16, 16)] = a
        return c

    lax.fori_loop(0, _RPT // 16, red, 0)
    pltpu.sync_copy(outb, out.at[cid, pl.ds(sid * _RPT, _RPT), 0])


@functools.cache
def _deg_agg_kernel():
    return pl.kernel(
        _deg_agg_body,
        mesh=plsc.VectorSubcoreMesh(core_axis_name="c",
                                    subcore_axis_name="s"),
        out_type=jax.ShapeDtypeStruct((_NC, _NP, 1), jnp.float32),
        scratch_types=[
            pltpu.VMEM((_DNIT, _DCH), jnp.int32),
            pltpu.VMEM((_DNIT, _DCH), jnp.int32),
            pltpu.VMEM((_NP,), jnp.float32),
            pltpu.VMEM((1, _NP), jnp.float32),
            pltpu.VMEM((_NS, _RPT), jnp.float32),
            pltpu.VMEM((_RPT,), jnp.float32),
            pltpu.VMEM_SHARED((_NS, _NP), jnp.float32),
        ],
    )


def _deg_agg(mvec, src4, dst4):
    return _deg_agg_kernel()(mvec, src4, dst4)


def _conv_body(acc_ref, dacc_ref, tab_ref, wrel_ref, brel_ref, wroot_ref,
               nm_ref, batch_ref, out_tab_ref, ro_ref):
    agg = acc_ref[0] + acc_ref[1]
    deg = dacc_ref[0] + dacc_ref[1]
    x = tab_ref[...]
    nm = nm_ref[...]
    mean = agg / jnp.maximum(deg, 1.0)
    h = (jnp.dot(mean, wrel_ref[...], preferred_element_type=jnp.float32)
         + brel_ref[...]
         + jnp.dot(x, wroot_ref[...], preferred_element_type=jnp.float32))
    h = jnp.maximum(h, 0.0) * nm
    out_tab_ref[...] = h
    b = batch_ref[...]
    g = lax.broadcasted_iota(jnp.int32, (_BLK, _B), 1).astype(jnp.float32)
    oh = (b == g).astype(jnp.float32)
    contrib = lax.dot_general(oh, h, (((0,), (0,)), ((), ())),
                              preferred_element_type=jnp.float32)

    @pl.when(pl.program_id(0) == 0)
    def _():
        ro_ref[...] = contrib

    @pl.when(pl.program_id(0) != 0)
    def _():
        ro_ref[...] += contrib


def _conv_step(acc, dacc, tab, wrel, brel, wroot, nm_col, batch_col):
    return pl.pallas_call(
        _conv_body,
        grid=(_G,),
        in_specs=[
            pl.BlockSpec((_NC, _BLK, _D), lambda i: (0, i, 0)),
            pl.BlockSpec((_NC, _BLK, 1), lambda i: (0, i, 0)),
            pl.BlockSpec((_BLK, _D), lambda i: (i, 0)),
            pl.BlockSpec((128, 128), lambda i: (0, 0)),
            pl.BlockSpec((1, 128), lambda i: (0, 0)),
            pl.BlockSpec((128, 128), lambda i: (0, 0)),
            pl.BlockSpec((_BLK, 1), lambda i: (i, 0)),
            pl.BlockSpec((_BLK, 1), lambda i: (i, 0)),
        ],
        out_specs=[
            pl.BlockSpec((_BLK, _D), lambda i: (i, 0)),
            pl.BlockSpec((_B, 128), lambda i: (0, 0)),
        ],
        out_shape=[
            jax.ShapeDtypeStruct((_NP, _D), jnp.float32),
            jax.ShapeDtypeStruct((_B, 128), jnp.float32),
        ],
    )(acc, dacc, tab, wrel, brel, wroot, nm_col, batch_col)


def _pool_body(acc_ref, dacc_ref, tab_ref, pwrel_ref, pbrel_ref, pwroot_ref,
               out_ref):
    agg = acc_ref[0] + acc_ref[1]
    deg = dacc_ref[0] + dacc_ref[1]
    x = tab_ref[...]
    mean = agg / jnp.maximum(deg, 1.0)
    s = (jnp.dot(mean, pwrel_ref[...], preferred_element_type=jnp.float32)
         + pbrel_ref[...]
         + jnp.dot(x, pwroot_ref[...], preferred_element_type=jnp.float32))
    out_ref[...] = jnp.tanh(s)


def _pool_score(acc, dacc, tab, pwrel, pbrel, pwroot):
    return pl.pallas_call(
        _pool_body,
        grid=(_G,),
        in_specs=[
            pl.BlockSpec((_NC, _BLK, _D), lambda i: (0, i, 0)),
            pl.BlockSpec((_NC, _BLK, 1), lambda i: (0, i, 0)),
            pl.BlockSpec((_BLK, _D), lambda i: (i, 0)),
            pl.BlockSpec((128, 128), lambda i: (0, 0)),
            pl.BlockSpec((1, 128), lambda i: (0, 0)),
            pl.BlockSpec((128, 128), lambda i: (0, 0)),
        ],
        out_specs=pl.BlockSpec((_BLK, 128), lambda i: (i, 0)),
        out_shape=jax.ShapeDtypeStruct((_NP, 128), jnp.float32),
    )(acc, dacc, tab, pwrel, pbrel, pwroot)


def _topk_body(scol_ref, srow_ref, bcol_ref, brow_ref, tab_ref,
               out_tab_ref, out_msk_ref):
    i = pl.program_id(0)
    si = scol_ref[pl.ds(i * _BLK, _BLK), :]
    bi = bcol_ref[pl.ds(i * _BLK, _BLK), :]
    ball = bcol_ref[...]
    gall = lax.broadcasted_iota(jnp.int32, (_NP, _B), 1).astype(jnp.float32)
    counts = jnp.sum((ball == gall).astype(jnp.float32), axis=0,
                     keepdims=True)
    krow = jnp.ceil(_RATIO * counts)
    ohi = (bi == lax.broadcasted_iota(jnp.int32, (_BLK, _B), 1)
           .astype(jnp.float32)).astype(jnp.float32)
    kv = lax.dot_general(ohi, krow, (((1,), (1,)), ((), ())),
                         preferred_element_type=jnp.float32)
    irow = (lax.broadcasted_iota(jnp.int32, (_BLK, _BLK), 0)
            .astype(jnp.float32) + (i * _BLK).astype(jnp.float32))

    def col_chunk(c, cnt):
        sj = srow_ref[:, pl.ds(c * _BLK, _BLK)]
        bj = brow_ref[:, pl.ds(c * _BLK, _BLK)]
        jcol = (lax.broadcasted_iota(jnp.int32, (_BLK, _BLK), 1)
                .astype(jnp.float32) + (c * _BLK).astype(jnp.float32))
        same = bi == bj
        ahead = (sj > si) | ((sj == si) & (jcol < irow))
        add = (same & ahead).astype(jnp.float32)
        return cnt + jnp.sum(add, axis=1, keepdims=True)

    cnt = lax.fori_loop(0, _G, col_chunk, jnp.zeros((_BLK, 1), jnp.float32))
    keep = (cnt < kv).astype(jnp.float32)
    out_tab_ref[...] = tab_ref[...] * si * keep
    out_msk_ref[...] = jnp.concatenate(
        [keep, jnp.zeros((_BLK, 127), jnp.float32)], axis=1)


def _topk_apply(scol, srow, bcol, brow, tab):
    return pl.pallas_call(
        _topk_body,
        grid=(_G,),
        in_specs=[
            pl.BlockSpec((_NP, 1), lambda i: (0, 0)),
            pl.BlockSpec((1, _NP), lambda i: (0, 0)),
            pl.BlockSpec((_NP, 1), lambda i: (0, 0)),
            pl.BlockSpec((1, _NP), lambda i: (0, 0)),
            pl.BlockSpec((_BLK, _D), lambda i: (i, 0)),
        ],
        out_specs=[
            pl.BlockSpec((_BLK, _D), lambda i: (i, 0)),
            pl.BlockSpec((_BLK, 128), lambda i: (i, 0)),
        ],
        out_shape=[
            jax.ShapeDtypeStruct((_NP, _D), jnp.float32),
            jax.ShapeDtypeStruct((_NP, 128), jnp.float32),
        ],
    )(scol, srow, bcol, brow, tab)


def _head_body(sc_ref, fc_ref, w1a_ref, w1b_ref, b1_ref, w2_ref, b2_ref,
               w3_ref, b3_ref, out_ref):
    sc = sc_ref[...]
    fc = fc_ref[...]
    h = jnp.maximum(
        jnp.dot(sc, w1a_ref[...], preferred_element_type=jnp.float32)
        + jnp.dot(fc, w1b_ref[...], preferred_element_type=jnp.float32)
        + b1_ref[...], 0.0)
    h = jnp.maximum(
        jnp.dot(h, w2_ref[...], preferred_element_type=jnp.float32)
        + b2_ref[...], 0.0)
    logits = (jnp.dot(h, w3_ref[...], preferred_element_type=jnp.float32)
              + b3_ref[...])

    def nrm(v):
        n2 = jnp.sum(v * v, axis=1, keepdims=True)
        return v / jnp.maximum(jnp.sqrt(n2), 1e-12)

    scn = nrm(sc)
    fcn = nrm(fc)
    dims = (((1,), (1,)), ((), ()))
    l_sc = lax.dot_general(scn, fcn, dims,
                           preferred_element_type=jnp.float32) / _TEMP
    l_fc = lax.dot_general(fcn, scn, dims,
                           preferred_element_type=jnp.float32) / _TEMP
    c_sc = lax.dot_general(scn, scn, dims,
                           preferred_element_type=jnp.float32) / _TEMP
    c_fc = lax.dot_general(fcn, fcn, dims,
                           preferred_element_type=jnp.float32) / _TEMP
    eye = (lax.broadcasted_iota(jnp.int32, (_B, _B), 0)
           == lax.broadcasted_iota(jnp.int32, (_B, _B), 1)
           ).astype(jnp.float32)
    neg_sc = c_sc * (1.0 - eye)
    neg_fc = c_fc * (1.0 - eye)

    def nce(l, neg):
        m = jnp.maximum(jnp.max(l, axis=1, keepdims=True),
                        jnp.max(neg, axis=1, keepdims=True))
        ssum = (jnp.sum(jnp.exp(l - m), axis=1, keepdims=True)
                + jnp.sum(jnp.exp(neg - m), axis=1, keepdims=True))
        lse = m + jnp.log(ssum)
        diag = jnp.sum(l * eye, axis=1, keepdims=True)
        return jnp.sum(lse - diag) / _B

    loss = (nce(l_sc, neg_sc) + nce(l_fc, neg_fc)) * 0.5
    lm = jnp.max(logits, axis=1, keepdims=True)
    lz = logits - lm
    lse2 = jnp.log(jnp.sum(jnp.exp(lz), axis=1, keepdims=True))
    out_ref[...] = lz - lse2 + loss


def _head(scf, fcf, w1a, w1b, b1, w2, b2, w3, b3):
    return pl.pallas_call(
        _head_body,
        out_shape=jax.ShapeDtypeStruct((_B, 10), jnp.float32),
    )(scf, fcf, w1a, w1b, b1, w2, b2, w3, b3)


def _pad_rows(xin):
    return jnp.concatenate(
        [xin, jnp.zeros((_NP - _N, _D), jnp.float32)], axis=0)


def _side(xin, ei, batch_col, p):
    src_e, dst_e = ei[0], ei[1]
    src4 = src_e.reshape(_NW, _DNIT, _DCH)
    dst4 = dst_e.reshape(_NW, _DNIT, _DCH)
    tab = _pad_rows(xin)
    ones_vec = jnp.ones((_NP,), jnp.float32)
    ones_col = jnp.ones((_NP, 1), jnp.float32)

    ones_tab = jnp.concatenate(
        [jnp.ones((_NP, 1), jnp.float32),
         jnp.zeros((_NP, 127), jnp.float32)], axis=1)
    dacc_full = _edge_agg(ones_tab, src_e, dst_e)[:, :, :1]

    wr, br, wt = p["conv0"]
    a = _edge_agg(tab, src_e, dst_e)
    tab, r0 = _conv_step(a, dacc_full, tab, wr, br.reshape(1, 128), wt,
                         ones_col, batch_col)
    wr, br, wt = p["convs"][0]
    a = _edge_agg(tab, src_e, dst_e)
    tab, r1 = _conv_step(a, dacc_full, tab, wr, br.reshape(1, 128), wt,
                         ones_col, batch_col)
    pw, pb, pwt = p["pool"]
    pwf = jnp.pad(pw, ((0, 0), (0, 127)))
    pbf = jnp.pad(pb.reshape(1, 1), ((0, 0), (0, 127)))
    pwtf = jnp.pad(pwt, ((0, 0), (0, 127)))
    a = _edge_agg(tab, src_e, dst_e)
    sfull = _pool_score(a, dacc_full, tab, pwf, pbf, pwtf)
    scol = sfull[:, :1]
    tab, msk_tab = _topk_apply(scol, scol.reshape(1, _NP), batch_col,
                               batch_col.reshape(1, _NP), tab)
    nm_col = msk_tab[:, :1]

    dacc_msk = _edge_agg(msk_tab, src_e, dst_e)[:, :, :1]

    wr, br, wt = p["convs"][1]
    a = _edge_agg(tab, src_e, dst_e)
    tab, r2 = _conv_step(a, dacc_msk, tab, wr, br.reshape(1, 128), wt,
                         nm_col, batch_col)
    wr, br, wt = p["convs"][2]
    a = _edge_agg(tab, src_e, dst_e)
    tab, r3 = _conv_step(a, dacc_msk, tab, wr, br.reshape(1, 128), wt,
                         nm_col, batch_col)
    return jnp.concatenate([r0, r1, r2, r3], axis=1)


def kernel(x, fc_x, params, edge_index, fc_edge_index, batch):
    batch_col = jnp.concatenate(
        [batch.astype(jnp.float32),
         jnp.full((_NP - _N,), -1.0, jnp.float32)]).reshape(_NP, 1)
    scf = _side(x, edge_index, batch_col, params["sc"])
    fcf = _side(fc_x, fc_edge_index, batch_col, params["fc"])
    w1, b1 = params["lin1"]
    w2, b2 = params["lin2"]
    w3, b3 = params["lin3"]
    return _head(scf, fcf, w1[:512], w1[512:], b1.reshape(1, 128),
                 w2, b2.reshape(1, 64), w3, b3.reshape(1, 10))
